# baseline (speedup 1.0000x reference)
import functools

import jax
import jax.numpy as jnp
from jax import lax
from jax.experimental import pallas as pl
from jax.experimental.pallas import tpu as pltpu
from jax.experimental.pallas import tpu_sc as plsc

B, N, E = 4, 1024, 16384
ED, ND, GD = 150, 100, 100
BOX = 6.0
HALF = BOX / 2.0
EPN = E // N
BN = B * N
BE = B * E
BN4 = BN * 4

NC, NS, L = 2, 16, 16
NW = NC * NS
EPT = BE // NW
NPT = BN // NW

TE = 1024
f32 = jnp.float32
i32 = jnp.int32


@functools.cache
def _mesh():
    return plsc.VectorSubcoreMesh(core_axis_name="c", subcore_axis_name="s",
                                  num_cores=NC, num_subcores=NS)


_SC_PARAMS = pltpu.CompilerParams(needs_layout_passes=False)


def _wid():
    return lax.axis_index("s") * NC + lax.axis_index("c")


def _sp(z):
    return jnp.maximum(z, 0.0) + jnp.log(1.0 + jnp.exp(-jnp.abs(z)))


def _sig(z, t):
    r = 1.0 / (1.0 + t)
    return jnp.where(z >= 0, r, 1.0 - r)


def _wrap_pbc(q):
    q = jnp.where(q >= HALF, q - BOX, q)
    return jnp.where(q < -HALF, q + BOX, q)


def _sc_gather_body(table, idxs5, idxr5, out, tab_v, is_v, ir_v, out_v):
    wid = _wid()
    base = wid * EPT
    pltpu.sync_copy(table, tab_v)
    pltpu.sync_copy(idxs5.at[pl.ds(base, EPT)], is_v)
    pltpu.sync_copy(idxr5.at[pl.ds(base, EPT)], ir_v)
    lane = lax.iota(i32, L)

    @plsc.parallel_loop(0, EPT // L, unroll=4)
    def _(i):
        ns = is_v[pl.ds(i * L, L)]
        nr = ir_v[pl.ds(i * L, L)]
        fs = [plsc.load_gather(tab_v, [ns + c]) for c in range(5)]
        fr = [plsc.load_gather(tab_v, [nr + c]) for c in range(5)]
        d0 = fs[0] - fr[0]
        d1 = fs[1] - fr[1]
        d0 = jnp.where(d0 > HALF, d0 - BOX, d0)
        d0 = jnp.where(d0 <= -HALF, d0 + BOX, d0)
        d1 = jnp.where(d1 > HALF, d1 - BOX, d1)
        d1 = jnp.where(d1 <= -HALF, d1 + BOX, d1)
        ebase = (i * L + lane) * 8
        ch = (fs[2], fs[3], fs[4], fr[2], fr[3], fr[4], d0, d1)
        for k in range(8):
            plsc.store_scatter(out_v, [ebase + k], ch[k])

    pltpu.sync_copy(out_v, out.at[pl.ds(base * 8, EPT * 8)])


@functools.cache
def _sc_gather():
    return pl.kernel(
        _sc_gather_body,
        out_type=jax.ShapeDtypeStruct((BE * 8,), f32),
        mesh=_mesh(),
        compiler_params=_SC_PARAMS,
        scratch_types=[
            pltpu.VMEM((BN * 5,), f32),
            pltpu.VMEM((EPT,), i32),
            pltpu.VMEM((EPT,), i32),
            pltpu.VMEM((EPT * 8,), f32),
        ],
    )


def _sc_scatter_body(pay, idxs4, idxr4, partials, pay_v, is_v, ir_v, acc_v):
    wid = _wid()
    base = wid * EPT
    pltpu.sync_copy(pay.at[pl.ds(base * 8, EPT * 8)], pay_v)
    pltpu.sync_copy(idxs4.at[pl.ds(base, EPT)], is_v)
    pltpu.sync_copy(idxr4.at[pl.ds(base, EPT)], ir_v)
    lane = lax.iota(i32, L)
    zv = jnp.zeros((L,), f32)

    @plsc.parallel_loop(0, BN4 // L, unroll=8)
    def _(j):
        acc_v[pl.ds(j * L, L)] = zv

    @plsc.parallel_loop(0, EPT // L, unroll=4)
    def _(i):
        ns = is_v[pl.ds(i * L, L)]
        nr = ir_v[pl.ds(i * L, L)]
        ebase = (i * L + lane) * 8
        for k in range(4):
            v = plsc.load_gather(pay_v, [ebase + k])
            plsc.addupdate_scatter(acc_v, [ns + k], v)
        for k in range(4):
            v = plsc.load_gather(pay_v, [ebase + 4 + k])
            plsc.addupdate_scatter(acc_v, [nr + k], v)

    pltpu.sync_copy(acc_v, partials.at[pl.ds(wid * BN4, BN4)])


@functools.cache
def _sc_scatter():
    return pl.kernel(
        _sc_scatter_body,
        out_type=jax.ShapeDtypeStruct((NW * BN4,), f32),
        mesh=_mesh(),
        compiler_params=_SC_PARAMS,
        scratch_types=[
            pltpu.VMEM((EPT * 8,), f32),
            pltpu.VMEM((EPT,), i32),
            pltpu.VMEM((EPT,), i32),
            pltpu.VMEM((BN4,), f32),
        ],
    )


def _accum_partials(partials, acc_v, tmp_v, nbase):
    C = NPT * 4
    pltpu.sync_copy(partials.at[pl.ds(nbase * 4, C)], acc_v)

    @pl.loop(1, NW)
    def _(w):
        pltpu.sync_copy(partials.at[pl.ds(w * BN4 + nbase * 4, C)], tmp_v)

        @pl.loop(0, C // L, unroll=8)
        def _(j):
            acc_v[pl.ds(j * L, L)] = acc_v[pl.ds(j * L, L)] + tmp_v[pl.ds(j * L, L)]


def _sc_update_body(partials, dnd, v0, mc, dtc, tab_out, k_out,
                    acc_v, tmp_v, dnd_v, v0_v, mc_v, dt_v, tab_v, k_v):
    wid = _wid()
    nbase = wid * NPT
    _accum_partials(partials, acc_v, tmp_v, nbase)
    pltpu.sync_copy(dnd.at[pl.ds(nbase * 2, NPT * 2)], dnd_v)
    pltpu.sync_copy(v0.at[pl.ds(nbase * 4, NPT * 4)], v0_v)
    pltpu.sync_copy(mc.at[pl.ds(nbase, NPT)], mc_v)
    pltpu.sync_copy(dtc.at[pl.ds(nbase, NPT)], dt_v)
    lane = lax.iota(i32, L)

    @pl.loop(0, NPT // L, unroll=2)
    def _(i):
        nl = i * L + lane
        g0 = plsc.load_gather(acc_v, [nl * 4])
        g1 = plsc.load_gather(acc_v, [nl * 4 + 1])
        g2 = plsc.load_gather(acc_v, [nl * 4 + 2])
        g3 = plsc.load_gather(acc_v, [nl * 4 + 3])
        d0 = plsc.load_gather(dnd_v, [nl * 2])
        d1 = plsc.load_gather(dnd_v, [nl * 2 + 1])
        kq1 = g2 + d0
        kq2 = g3 + d1
        kp1 = -g0
        kp2 = -g1
        dtv = dt_v[pl.ds(i * L, L)]
        mcv = mc_v[pl.ds(i * L, L)]
        q1 = _wrap_pbc(plsc.load_gather(v0_v, [nl * 4]) + kq1 * dtv)
        q2 = _wrap_pbc(plsc.load_gather(v0_v, [nl * 4 + 1]) + kq2 * dtv)
        p1 = plsc.load_gather(v0_v, [nl * 4 + 2]) + kp1 * dtv
        p2 = plsc.load_gather(v0_v, [nl * 4 + 3]) + kp2 * dtv
        plsc.store_scatter(tab_v, [nl * 5], q1)
        plsc.store_scatter(tab_v, [nl * 5 + 1], q2)
        plsc.store_scatter(tab_v, [nl * 5 + 2], mcv)
        plsc.store_scatter(tab_v, [nl * 5 + 3], p1)
        plsc.store_scatter(tab_v, [nl * 5 + 4], p2)
        plsc.store_scatter(k_v, [nl * 4], kq1)
        plsc.store_scatter(k_v, [nl * 4 + 1], kq2)
        plsc.store_scatter(k_v, [nl * 4 + 2], kp1)
        plsc.store_scatter(k_v, [nl * 4 + 3], kp2)

    pltpu.sync_copy(tab_v, tab_out.at[pl.ds(nbase * 5, NPT * 5)])
    pltpu.sync_copy(k_v, k_out.at[pl.ds(nbase * 4, NPT * 4)])


@functools.cache
def _sc_update():
    return pl.kernel(
        _sc_update_body,
        out_type=(jax.ShapeDtypeStruct((BN * 5,), f32),
                  jax.ShapeDtypeStruct((BN4,), f32)),
        mesh=_mesh(),
        compiler_params=_SC_PARAMS,
        scratch_types=[
            pltpu.VMEM((NPT * 4,), f32),
            pltpu.VMEM((NPT * 4,), f32),
            pltpu.VMEM((NPT * 2,), f32),
            pltpu.VMEM((NPT * 4,), f32),
            pltpu.VMEM((NPT,), f32),
            pltpu.VMEM((NPT,), f32),
            pltpu.VMEM((NPT * 5,), f32),
            pltpu.VMEM((NPT * 4,), f32),
        ],
    )


def _sc_final_body(partials, dnd, v0, invm, dt6, k1, k2, k3, out,
                   acc_v, tmp_v, dnd_v, v0_v, im_v, dt_v, k1_v, k2_v, k3_v, o_v):
    wid = _wid()
    nbase = wid * NPT
    _accum_partials(partials, acc_v, tmp_v, nbase)
    pltpu.sync_copy(dnd.at[pl.ds(nbase * 2, NPT * 2)], dnd_v)
    pltpu.sync_copy(v0.at[pl.ds(nbase * 4, NPT * 4)], v0_v)
    pltpu.sync_copy(invm.at[pl.ds(nbase, NPT)], im_v)
    pltpu.sync_copy(dt6.at[pl.ds(nbase, NPT)], dt_v)
    pltpu.sync_copy(k1.at[pl.ds(nbase * 4, NPT * 4)], k1_v)
    pltpu.sync_copy(k2.at[pl.ds(nbase * 4, NPT * 4)], k2_v)
    pltpu.sync_copy(k3.at[pl.ds(nbase * 4, NPT * 4)], k3_v)
    lane = lax.iota(i32, L)

    @pl.loop(0, NPT // L, unroll=2)
    def _(i):
        nl = i * L + lane
        g0 = plsc.load_gather(acc_v, [nl * 4])
        g1 = plsc.load_gather(acc_v, [nl * 4 + 1])
        g2 = plsc.load_gather(acc_v, [nl * 4 + 2])
        g3 = plsc.load_gather(acc_v, [nl * 4 + 3])
        d0 = plsc.load_gather(dnd_v, [nl * 2])
        d1 = plsc.load_gather(dnd_v, [nl * 2 + 1])
        k4 = (g2 + d0, g3 + d1, -g0, -g1)
        dtv = dt_v[pl.ds(i * L, L)]
        imv = im_v[pl.ds(i * L, L)]
        for c in range(4):
            k1c = plsc.load_gather(k1_v, [nl * 4 + c])
            k2c = plsc.load_gather(k2_v, [nl * 4 + c])
            k3c = plsc.load_gather(k3_v, [nl * 4 + c])
            dy = dtv * (k1c + 2.0 * k2c + 2.0 * k3c + k4[c])
            v = plsc.load_gather(v0_v, [nl * 4 + c]) + dy
            if c < 2:
                v = _wrap_pbc(v)
            else:
                v = v * imv
            plsc.store_scatter(o_v, [nl * 4 + c], v)

    pltpu.sync_copy(o_v, out.at[pl.ds(nbase * 4, NPT * 4)])


@functools.cache
def _sc_final():
    return pl.kernel(
        _sc_final_body,
        out_type=jax.ShapeDtypeStruct((BN4,), f32),
        mesh=_mesh(),
        compiler_params=_SC_PARAMS,
        scratch_types=[
            pltpu.VMEM((NPT * 4,), f32),
            pltpu.VMEM((NPT * 4,), f32),
            pltpu.VMEM((NPT * 2,), f32),
            pltpu.VMEM((NPT * 4,), f32),
            pltpu.VMEM((NPT,), f32),
            pltpu.VMEM((NPT,), f32),
            pltpu.VMEM((NPT * 4,), f32),
            pltpu.VMEM((NPT * 4,), f32),
            pltpu.VMEM((NPT * 4,), f32),
            pltpu.VMEM((NPT * 4,), f32),
        ],
    )


NEB = E // TE
PH_NODE = NEB
NSTEP = 2 * NEB + 1


def _stage_body(ecat_ref, w1_ref, b1_ref, w2_ref, b2_ref, w2T_ref, w1TM_ref,
                wn1np_ref, wn1agg_ref, bn1_ref, wn2_ref, bn2_ref, wn3_ref,
                bn3_ref, wg1v_ref, wg1e_ref, bg1_ref, wg2_ref, bg2_ref,
                woutT_ref, wg2T_ref, wg1vT_ref, wg1eT_ref, wn3T_ref, wn2T_ref,
                wn1Tagg_ref, wn1Tmom_ref, vnp_ref,
                pay_ref, dnd_ref,
                a1_s, en_s, agg_s, dagg_s, deng_s):
    j = pl.program_id(1)

    @pl.when(j < PH_NODE)
    def _fwd():
        x = ecat_ref[0]
        z1 = jnp.dot(x, w1_ref[...], preferred_element_type=f32) + b1_ref[...]
        a1 = _sp(z1)
        z2 = jnp.dot(a1, w2_ref[...], preferred_element_type=f32) + b2_ref[...]
        en = _sp(z2)
        a1_s[j] = a1
        en_s[j] = en
        agg_s[j] = en.reshape(TE // EPN, EPN, ED).sum(axis=1)

    @pl.when(j == PH_NODE)
    def _node():
        vnp = vnp_ref[0][:, 2:5]
        agg = agg_s[...].reshape(N, ED)
        zn1 = (jnp.dot(vnp, wn1np_ref[...], preferred_element_type=f32)
               + jnp.dot(agg, wn1agg_ref[...], preferred_element_type=f32)
               + bn1_ref[...])
        t1 = jnp.exp(-jnp.abs(zn1))
        an1 = jnp.maximum(zn1, 0.0) + jnp.log(1.0 + t1)
        zn2 = (jnp.dot(an1, wn2_ref[...], preferred_element_type=f32)
               + bn2_ref[...])
        t2 = jnp.exp(-jnp.abs(zn2))
        an2 = jnp.maximum(zn2, 0.0) + jnp.log(1.0 + t2)
        zn3 = (jnp.dot(an2, wn3_ref[...], preferred_element_type=f32)
               + bn3_ref[...])
        t3 = jnp.exp(-jnp.abs(zn3))
        vn = jnp.maximum(zn3, 0.0) + jnp.log(1.0 + t3)
        vsum = vn.sum(axis=0, keepdims=True)
        esum = agg.sum(axis=0, keepdims=True)
        zg1 = (jnp.dot(vsum, wg1v_ref[...], preferred_element_type=f32)
               + jnp.dot(esum, wg1e_ref[...], preferred_element_type=f32)
               + bg1_ref[...])
        tg1 = jnp.exp(-jnp.abs(zg1))
        u1 = jnp.maximum(zg1, 0.0) + jnp.log(1.0 + tg1)
        zg2 = (jnp.dot(u1, wg2_ref[...], preferred_element_type=f32)
               + bg2_ref[...])
        tg2 = jnp.exp(-jnp.abs(zg2))
        dzg2 = woutT_ref[...] * _sig(zg2, tg2)
        du1 = jnp.dot(dzg2, wg2T_ref[...], preferred_element_type=f32)
        dzg1 = du1 * _sig(zg1, tg1)
        dgv = jnp.dot(dzg1, wg1vT_ref[...], preferred_element_type=f32)
        dge = jnp.dot(dzg1, wg1eT_ref[...], preferred_element_type=f32)
        dzn3 = dgv * _sig(zn3, t3)
        dan2 = jnp.dot(dzn3, wn3T_ref[...], preferred_element_type=f32)
        dzn2 = dan2 * _sig(zn2, t2)
        dan1 = jnp.dot(dzn2, wn2T_ref[...], preferred_element_type=f32)
        dzn1 = dan1 * _sig(zn1, t1)
        dagg_s[...] = jnp.dot(dzn1, wn1Tagg_ref[...],
                              preferred_element_type=f32).reshape(
                                  NEB, TE // EPN, ED)
        dnd_ref[0] = jnp.dot(dzn1, wn1Tmom_ref[...],
                             preferred_element_type=f32)
        deng_s[...] = dge

    @pl.when(j > PH_NODE)
    def _bwd():
        k = j - (PH_NODE + 1)
        a1 = a1_s[k]
        en = en_s[k]
        s1 = 1.0 - jnp.exp(-a1)
        s2 = 1.0 - jnp.exp(-en)
        rep = jnp.broadcast_to(dagg_s[k][:, None, :], (TE // EPN, EPN, ED))
        den = rep.reshape(TE, ED) + deng_s[...]
        dz2 = den * s2
        da1 = jnp.dot(dz2, w2T_ref[...], preferred_element_type=f32)
        dz1 = da1 * s1
        pay_ref[0] = jnp.dot(dz1, w1TM_ref[...], preferred_element_type=f32)


def _ecat_index(b, j):
    jj = jnp.where(j < PH_NODE, j, j - (PH_NODE + 1))
    jj = jnp.maximum(jj, 0)
    return (b, jj, 0)


def _pay_index(b, j):
    k = jnp.maximum(j - (PH_NODE + 1), 0)
    return (b, k, 0)


def _tc_stage(ecat, vnp, w):
    full = lambda shape: pl.BlockSpec(shape, lambda b, j: (0,) * len(shape))
    return pl.pallas_call(
        _stage_body,
        grid=(B, NSTEP),
        in_specs=[
            pl.BlockSpec((1, TE, 8), _ecat_index),
            full((8, ED)), full((1, ED)), full((ED, ED)), full((1, ED)),
            full((ED, ED)), full((ED, 8)),
            full((3, ND)), full((ED, ND)), full((1, ND)),
            full((ND, ND)), full((1, ND)), full((ND, ND)), full((1, ND)),
            full((ND, GD)), full((ED, GD)), full((1, GD)),
            full((GD, GD)), full((1, GD)),
            full((1, GD)), full((GD, GD)), full((GD, ND)), full((GD, ED)),
            full((ND, ND)), full((ND, ND)), full((ND, ED)), full((ND, 2)),
            pl.BlockSpec((1, N, 5), lambda b, j: (b, 0, 0)),
        ],
        out_specs=[
            pl.BlockSpec((1, TE, 8), _pay_index),
            pl.BlockSpec((1, N, 2), lambda b, j: (b, 0, 0)),
        ],
        out_shape=[
            jax.ShapeDtypeStruct((B, E, 8), f32),
            jax.ShapeDtypeStruct((B, N, 2), f32),
        ],
        scratch_shapes=[
            pltpu.VMEM((NEB, TE, ED), f32),
            pltpu.VMEM((NEB, TE, ED), f32),
            pltpu.VMEM((NEB, TE // EPN, ED), f32),
            pltpu.VMEM((NEB, TE // EPN, ED), f32),
            pltpu.VMEM((1, ED), f32),
        ],
    )(ecat, w["e1"], w["be1"], w["e2"], w["be2"], w["e2T"], w["e1TM"],
      w["n1np"], w["n1agg"], w["bn1"], w["n2"], w["bn2"], w["n3"], w["bn3"],
      w["g1v"], w["g1e"], w["bg1"], w["g2"], w["bg2"],
      w["outT"], w["g2T"], w["g1vT"], w["g1eT"], w["n3T"], w["n2T"],
      w["n1Tagg"], w["n1Tmom"], vnp)


def _prep_weights(params):
    w = {}
    w["e1"] = params["e1"]["W"]
    w["be1"] = params["e1"]["b"].reshape(1, ED)
    w["e2"] = params["e2"]["W"]
    w["be2"] = params["e2"]["b"].reshape(1, ED)
    w["e2T"] = params["e2"]["W"].T
    M = jnp.zeros((8, 8), f32)
    M = M.at[6, 0].set(1.0).at[7, 1].set(1.0)
    M = M.at[1, 2].set(1.0).at[2, 3].set(1.0)
    M = M.at[6, 4].set(-1.0).at[7, 5].set(-1.0)
    M = M.at[4, 6].set(1.0).at[5, 7].set(1.0)
    w["e1TM"] = params["e1"]["W"].T @ M
    wn1 = params["n1"]["W"]
    w["n1np"] = wn1[:3]
    w["n1agg"] = wn1[3:]
    w["bn1"] = params["n1"]["b"].reshape(1, ND)
    w["n2"] = params["n2"]["W"]
    w["bn2"] = params["n2"]["b"].reshape(1, ND)
    w["n3"] = params["n3"]["W"]
    w["bn3"] = params["n3"]["b"].reshape(1, ND)
    w["n2T"] = params["n2"]["W"].T
    w["n3T"] = params["n3"]["W"].T
    w["n1Tagg"] = wn1[3:].T
    w["n1Tmom"] = wn1[1:3].T
    wg1 = params["g1"]["W"]
    w["g1v"] = wg1[:ND]
    w["g1e"] = wg1[ND:]
    w["bg1"] = params["g1"]["b"].reshape(1, GD)
    w["g2"] = params["g2"]["W"]
    w["bg2"] = params["g2"]["b"].reshape(1, GD)
    w["g2T"] = params["g2"]["W"].T
    w["g1vT"] = wg1[:ND].T
    w["g1eT"] = wg1[ND:].T
    w["outT"] = params["out"]["W"][:, 0].reshape(1, GD)
    return w


def kernel(state, R_s, R_r, dt, params):
    w = _prep_weights(params)
    mc = state[:, :, 0]
    m3 = state[:, :, 0:1]
    q = state[:, :, 1:3]
    mom = state[:, :, 3:5] * m3
    v0f = jnp.concatenate([q, mom], axis=2).reshape(-1)
    table0 = jnp.concatenate([q, m3, mom], axis=2).reshape(-1)
    mcf = mc.reshape(-1)
    invmf = (1.0 / mc).reshape(-1)
    dth = (dt * 0.5).reshape(-1)
    dtf = dt.reshape(-1)
    dt6 = (dt / 6.0).reshape(-1)
    boff = (jnp.arange(B, dtype=i32) * N)[:, None]
    gs = (R_s.astype(i32) + boff).reshape(-1)
    gr = (R_r.astype(i32) + boff).reshape(-1)
    gs5 = gs * 5
    gr5 = gr * 5
    gs4 = gs * 4
    gr4 = gr * 4

    def stage(table):
        ecat = _sc_gather()(table, gs5, gr5).reshape(B, E, 8)
        pay, dnd = _tc_stage(ecat, table.reshape(B, N, 5), w)
        partials = _sc_scatter()(pay.reshape(-1), gs4, gr4)
        return partials, dnd.reshape(-1)

    p1, dnd1 = stage(table0)
    tab2, k1 = _sc_update()(p1, dnd1, v0f, mcf, dth)
    p2, dnd2 = stage(tab2)
    tab3, k2 = _sc_update()(p2, dnd2, v0f, mcf, dth)
    p3, dnd3 = stage(tab3)
    tab4, k3 = _sc_update()(p3, dnd3, v0f, mcf, dtf)
    p4, dnd4 = stage(tab4)
    outf = _sc_final()(p4, dnd4, v0f, invmf, dt6, k1, k2, k3)
    return outf.reshape(B, N, 4)


# speedup vs baseline: 1.1382x; 1.1382x over previous
import functools

import jax
import jax.numpy as jnp
from jax import lax
from jax.experimental import pallas as pl
from jax.experimental.pallas import tpu as pltpu
from jax.experimental.pallas import tpu_sc as plsc

B, N, E = 4, 1024, 16384
ED, ND, GD = 150, 100, 100
BOX = 6.0
HALF = BOX / 2.0
EPN = E // N
BN = B * N
BE = B * E
BN4 = BN * 4

NC, NS, L = 2, 16, 16
NW = NC * NS
EPT = BE // NW
NPT = BN // NW

TE = 1024
f32 = jnp.float32
i32 = jnp.int32


@functools.cache
def _mesh():
    return plsc.VectorSubcoreMesh(core_axis_name="c", subcore_axis_name="s",
                                  num_cores=NC, num_subcores=NS)


_SC_PARAMS = pltpu.CompilerParams(needs_layout_passes=False)


def _wid():
    return lax.axis_index("s") * NC + lax.axis_index("c")


def _sp(z):
    return jnp.maximum(z, 0.0) + jnp.log(1.0 + jnp.exp(-jnp.abs(z)))


def _sig(z, t):
    r = 1.0 / (1.0 + t)
    return jnp.where(z >= 0, r, 1.0 - r)


def _wrap_pbc(q):
    q = jnp.where(q >= HALF, q - BOX, q)
    return jnp.where(q < -HALF, q + BOX, q)


def _sc_gather_body(table, idxs5, idxr5, out, tab_v, is_v, ir_v, out_v):
    wid = _wid()
    base = wid * EPT
    pltpu.sync_copy(table, tab_v)
    pltpu.sync_copy(idxs5.at[pl.ds(base, EPT)], is_v)
    pltpu.sync_copy(idxr5.at[pl.ds(base, EPT)], ir_v)
    lane = lax.iota(i32, L)

    @plsc.parallel_loop(0, EPT // L, unroll=4)
    def _(i):
        ns = is_v[pl.ds(i * L, L)]
        nr = ir_v[pl.ds(i * L, L)]
        fs = [plsc.load_gather(tab_v, [ns + c]) for c in range(5)]
        fr = [plsc.load_gather(tab_v, [nr + c]) for c in range(5)]
        d0 = fs[0] - fr[0]
        d1 = fs[1] - fr[1]
        d0 = jnp.where(d0 > HALF, d0 - BOX, d0)
        d0 = jnp.where(d0 <= -HALF, d0 + BOX, d0)
        d1 = jnp.where(d1 > HALF, d1 - BOX, d1)
        d1 = jnp.where(d1 <= -HALF, d1 + BOX, d1)
        ebase = (i * L + lane) * 8
        ch = (fs[2], fs[3], fs[4], fr[2], fr[3], fr[4], d0, d1)
        for k in range(8):
            plsc.store_scatter(out_v, [ebase + k], ch[k])

    pltpu.sync_copy(out_v, out.at[pl.ds(base * 8, EPT * 8)])


@functools.cache
def _sc_gather():
    return pl.kernel(
        _sc_gather_body,
        out_type=jax.ShapeDtypeStruct((BE * 8,), f32),
        mesh=_mesh(),
        compiler_params=_SC_PARAMS,
        scratch_types=[
            pltpu.VMEM((BN * 5,), f32),
            pltpu.VMEM((EPT,), i32),
            pltpu.VMEM((EPT,), i32),
            pltpu.VMEM((EPT * 8,), f32),
        ],
    )


def _sc_scatter_body(payT, idxs4, idxr4, partials, pay_v, is_v, ir_v, acc_v):
    wid = _wid()
    base = wid * EPT
    pltpu.sync_copy(payT.at[:, pl.ds(base, EPT)], pay_v)
    pltpu.sync_copy(idxs4.at[pl.ds(base, EPT)], is_v)
    pltpu.sync_copy(idxr4.at[pl.ds(base, EPT)], ir_v)
    lane = lax.iota(i32, L)
    zv = jnp.zeros((L,), f32)

    @plsc.parallel_loop(0, BN4 // L, unroll=8)
    def _(j):
        acc_v[pl.ds(j * L, L)] = zv

    @plsc.parallel_loop(0, EPT // L, unroll=4)
    def _(i):
        ns = is_v[pl.ds(i * L, L)]
        nr = ir_v[pl.ds(i * L, L)]
        for k in range(4):
            plsc.addupdate_scatter(acc_v, [ns + k], pay_v[k, pl.ds(i * L, L)])
        for k in range(4):
            plsc.addupdate_scatter(acc_v, [nr + k],
                                   pay_v[4 + k, pl.ds(i * L, L)])

    pltpu.sync_copy(acc_v, partials.at[pl.ds(wid * BN4, BN4)])


@functools.cache
def _sc_scatter():
    return pl.kernel(
        _sc_scatter_body,
        out_type=jax.ShapeDtypeStruct((NW * BN4,), f32),
        mesh=_mesh(),
        compiler_params=_SC_PARAMS,
        scratch_types=[
            pltpu.VMEM((8, EPT), f32),
            pltpu.VMEM((EPT,), i32),
            pltpu.VMEM((EPT,), i32),
            pltpu.VMEM((BN4,), f32),
        ],
    )


def _accum_partials(partials, acc_v, tmp_v, nbase):
    C = NPT * 4
    pltpu.sync_copy(partials.at[pl.ds(nbase * 4, C)], acc_v)

    @pl.loop(1, NW)
    def _(w):
        pltpu.sync_copy(partials.at[pl.ds(w * BN4 + nbase * 4, C)], tmp_v)

        @pl.loop(0, C // L, unroll=8)
        def _(j):
            acc_v[pl.ds(j * L, L)] = acc_v[pl.ds(j * L, L)] + tmp_v[pl.ds(j * L, L)]


def _sc_update_body(partials, dnd, v0, mc, dtc, tab_out, k_out,
                    acc_v, tmp_v, dnd_v, v0_v, mc_v, dt_v, tab_v, k_v):
    wid = _wid()
    nbase = wid * NPT
    _accum_partials(partials, acc_v, tmp_v, nbase)
    pltpu.sync_copy(dnd.at[pl.ds(nbase * 2, NPT * 2)], dnd_v)
    pltpu.sync_copy(v0.at[pl.ds(nbase * 4, NPT * 4)], v0_v)
    pltpu.sync_copy(mc.at[pl.ds(nbase, NPT)], mc_v)
    pltpu.sync_copy(dtc.at[pl.ds(nbase, NPT)], dt_v)
    lane = lax.iota(i32, L)

    @pl.loop(0, NPT // L, unroll=2)
    def _(i):
        nl = i * L + lane
        g0 = plsc.load_gather(acc_v, [nl * 4])
        g1 = plsc.load_gather(acc_v, [nl * 4 + 1])
        g2 = plsc.load_gather(acc_v, [nl * 4 + 2])
        g3 = plsc.load_gather(acc_v, [nl * 4 + 3])
        d0 = plsc.load_gather(dnd_v, [nl * 2])
        d1 = plsc.load_gather(dnd_v, [nl * 2 + 1])
        kq1 = g2 + d0
        kq2 = g3 + d1
        kp1 = -g0
        kp2 = -g1
        dtv = dt_v[pl.ds(i * L, L)]
        mcv = mc_v[pl.ds(i * L, L)]
        q1 = _wrap_pbc(plsc.load_gather(v0_v, [nl * 4]) + kq1 * dtv)
        q2 = _wrap_pbc(plsc.load_gather(v0_v, [nl * 4 + 1]) + kq2 * dtv)
        p1 = plsc.load_gather(v0_v, [nl * 4 + 2]) + kp1 * dtv
        p2 = plsc.load_gather(v0_v, [nl * 4 + 3]) + kp2 * dtv
        plsc.store_scatter(tab_v, [nl * 5], q1)
        plsc.store_scatter(tab_v, [nl * 5 + 1], q2)
        plsc.store_scatter(tab_v, [nl * 5 + 2], mcv)
        plsc.store_scatter(tab_v, [nl * 5 + 3], p1)
        plsc.store_scatter(tab_v, [nl * 5 + 4], p2)
        plsc.store_scatter(k_v, [nl * 4], kq1)
        plsc.store_scatter(k_v, [nl * 4 + 1], kq2)
        plsc.store_scatter(k_v, [nl * 4 + 2], kp1)
        plsc.store_scatter(k_v, [nl * 4 + 3], kp2)

    pltpu.sync_copy(tab_v, tab_out.at[pl.ds(nbase * 5, NPT * 5)])
    pltpu.sync_copy(k_v, k_out.at[pl.ds(nbase * 4, NPT * 4)])


@functools.cache
def _sc_update():
    return pl.kernel(
        _sc_update_body,
        out_type=(jax.ShapeDtypeStruct((BN * 5,), f32),
                  jax.ShapeDtypeStruct((BN4,), f32)),
        mesh=_mesh(),
        compiler_params=_SC_PARAMS,
        scratch_types=[
            pltpu.VMEM((NPT * 4,), f32),
            pltpu.VMEM((NPT * 4,), f32),
            pltpu.VMEM((NPT * 2,), f32),
            pltpu.VMEM((NPT * 4,), f32),
            pltpu.VMEM((NPT,), f32),
            pltpu.VMEM((NPT,), f32),
            pltpu.VMEM((NPT * 5,), f32),
            pltpu.VMEM((NPT * 4,), f32),
        ],
    )


def _sc_final_body(partials, dnd, v0, invm, dt6, k1, k2, k3, out,
                   acc_v, tmp_v, dnd_v, v0_v, im_v, dt_v, k1_v, k2_v, k3_v, o_v):
    wid = _wid()
    nbase = wid * NPT
    _accum_partials(partials, acc_v, tmp_v, nbase)
    pltpu.sync_copy(dnd.at[pl.ds(nbase * 2, NPT * 2)], dnd_v)
    pltpu.sync_copy(v0.at[pl.ds(nbase * 4, NPT * 4)], v0_v)
    pltpu.sync_copy(invm.at[pl.ds(nbase, NPT)], im_v)
    pltpu.sync_copy(dt6.at[pl.ds(nbase, NPT)], dt_v)
    pltpu.sync_copy(k1.at[pl.ds(nbase * 4, NPT * 4)], k1_v)
    pltpu.sync_copy(k2.at[pl.ds(nbase * 4, NPT * 4)], k2_v)
    pltpu.sync_copy(k3.at[pl.ds(nbase * 4, NPT * 4)], k3_v)
    lane = lax.iota(i32, L)

    @pl.loop(0, NPT // L, unroll=2)
    def _(i):
        nl = i * L + lane
        g0 = plsc.load_gather(acc_v, [nl * 4])
        g1 = plsc.load_gather(acc_v, [nl * 4 + 1])
        g2 = plsc.load_gather(acc_v, [nl * 4 + 2])
        g3 = plsc.load_gather(acc_v, [nl * 4 + 3])
        d0 = plsc.load_gather(dnd_v, [nl * 2])
        d1 = plsc.load_gather(dnd_v, [nl * 2 + 1])
        k4 = (g2 + d0, g3 + d1, -g0, -g1)
        dtv = dt_v[pl.ds(i * L, L)]
        imv = im_v[pl.ds(i * L, L)]
        for c in range(4):
            k1c = plsc.load_gather(k1_v, [nl * 4 + c])
            k2c = plsc.load_gather(k2_v, [nl * 4 + c])
            k3c = plsc.load_gather(k3_v, [nl * 4 + c])
            dy = dtv * (k1c + 2.0 * k2c + 2.0 * k3c + k4[c])
            v = plsc.load_gather(v0_v, [nl * 4 + c]) + dy
            if c < 2:
                v = _wrap_pbc(v)
            else:
                v = v * imv
            plsc.store_scatter(o_v, [nl * 4 + c], v)

    pltpu.sync_copy(o_v, out.at[pl.ds(nbase * 4, NPT * 4)])


@functools.cache
def _sc_final():
    return pl.kernel(
        _sc_final_body,
        out_type=jax.ShapeDtypeStruct((BN4,), f32),
        mesh=_mesh(),
        compiler_params=_SC_PARAMS,
        scratch_types=[
            pltpu.VMEM((NPT * 4,), f32),
            pltpu.VMEM((NPT * 4,), f32),
            pltpu.VMEM((NPT * 2,), f32),
            pltpu.VMEM((NPT * 4,), f32),
            pltpu.VMEM((NPT,), f32),
            pltpu.VMEM((NPT,), f32),
            pltpu.VMEM((NPT * 4,), f32),
            pltpu.VMEM((NPT * 4,), f32),
            pltpu.VMEM((NPT * 4,), f32),
            pltpu.VMEM((NPT * 4,), f32),
        ],
    )


NEB = E // TE
PH_NODE = NEB
NSTEP = 2 * NEB + 1


def _stage_body(ecat_ref, w1_ref, b1_ref, w2_ref, b2_ref, w2T_ref, w1TM_ref,
                wn1np_ref, wn1agg_ref, bn1_ref, wn2_ref, bn2_ref, wn3_ref,
                bn3_ref, wg1v_ref, wg1e_ref, bg1_ref, wg2_ref, bg2_ref,
                woutT_ref, wg2T_ref, wg1vT_ref, wg1eT_ref, wn3T_ref, wn2T_ref,
                wn1Tagg_ref, wn1Tmom_ref, vnp_ref,
                payT_ref, dnd_ref,
                a1_s, en_s, agg_s, dagg_s, deng_s):
    j = pl.program_id(1)

    @pl.when(j < PH_NODE)
    def _fwd():
        x = ecat_ref[0]
        z1 = jnp.dot(x, w1_ref[...], preferred_element_type=f32) + b1_ref[...]
        a1 = _sp(z1)
        z2 = jnp.dot(a1, w2_ref[...], preferred_element_type=f32) + b2_ref[...]
        en = _sp(z2)
        a1_s[j] = a1
        en_s[j] = en
        agg_s[j] = en.reshape(TE // EPN, EPN, ED).sum(axis=1)

    @pl.when(j == PH_NODE)
    def _node():
        vnp = vnp_ref[0][:, 2:5]
        agg = agg_s[...].reshape(N, ED)
        zn1 = (jnp.dot(vnp, wn1np_ref[...], preferred_element_type=f32)
               + jnp.dot(agg, wn1agg_ref[...], preferred_element_type=f32)
               + bn1_ref[...])
        t1 = jnp.exp(-jnp.abs(zn1))
        an1 = jnp.maximum(zn1, 0.0) + jnp.log(1.0 + t1)
        zn2 = (jnp.dot(an1, wn2_ref[...], preferred_element_type=f32)
               + bn2_ref[...])
        t2 = jnp.exp(-jnp.abs(zn2))
        an2 = jnp.maximum(zn2, 0.0) + jnp.log(1.0 + t2)
        zn3 = (jnp.dot(an2, wn3_ref[...], preferred_element_type=f32)
               + bn3_ref[...])
        t3 = jnp.exp(-jnp.abs(zn3))
        vn = jnp.maximum(zn3, 0.0) + jnp.log(1.0 + t3)
        vsum = vn.sum(axis=0, keepdims=True)
        esum = agg.sum(axis=0, keepdims=True)
        zg1 = (jnp.dot(vsum, wg1v_ref[...], preferred_element_type=f32)
               + jnp.dot(esum, wg1e_ref[...], preferred_element_type=f32)
               + bg1_ref[...])
        tg1 = jnp.exp(-jnp.abs(zg1))
        u1 = jnp.maximum(zg1, 0.0) + jnp.log(1.0 + tg1)
        zg2 = (jnp.dot(u1, wg2_ref[...], preferred_element_type=f32)
               + bg2_ref[...])
        tg2 = jnp.exp(-jnp.abs(zg2))
        dzg2 = woutT_ref[...] * _sig(zg2, tg2)
        du1 = jnp.dot(dzg2, wg2T_ref[...], preferred_element_type=f32)
        dzg1 = du1 * _sig(zg1, tg1)
        dgv = jnp.dot(dzg1, wg1vT_ref[...], preferred_element_type=f32)
        dge = jnp.dot(dzg1, wg1eT_ref[...], preferred_element_type=f32)
        dzn3 = dgv * _sig(zn3, t3)
        dan2 = jnp.dot(dzn3, wn3T_ref[...], preferred_element_type=f32)
        dzn2 = dan2 * _sig(zn2, t2)
        dan1 = jnp.dot(dzn2, wn2T_ref[...], preferred_element_type=f32)
        dzn1 = dan1 * _sig(zn1, t1)
        dagg_s[...] = jnp.dot(dzn1, wn1Tagg_ref[...],
                              preferred_element_type=f32).reshape(
                                  NEB, TE // EPN, ED)
        dnd_ref[0] = jnp.dot(dzn1, wn1Tmom_ref[...],
                             preferred_element_type=f32)
        deng_s[...] = dge

    @pl.when(j > PH_NODE)
    def _bwd():
        k = j - (PH_NODE + 1)
        a1 = a1_s[k]
        en = en_s[k]
        s1 = 1.0 - jnp.exp(-a1)
        s2 = 1.0 - jnp.exp(-en)
        rep = jnp.broadcast_to(dagg_s[k][:, None, :], (TE // EPN, EPN, ED))
        den = rep.reshape(TE, ED) + deng_s[...]
        dz2 = den * s2
        da1 = jnp.dot(dz2, w2T_ref[...], preferred_element_type=f32)
        dz1 = da1 * s1
        pay = jnp.dot(dz1, w1TM_ref[...], preferred_element_type=f32)
        payT_ref[...] = pay.T


def _ecat_index(b, j):
    jj = jnp.where(j < PH_NODE, j, j - (PH_NODE + 1))
    jj = jnp.maximum(jj, 0)
    return (b, jj, 0)


def _payT_index(b, j):
    k = jnp.maximum(j - (PH_NODE + 1), 0)
    return (0, b * NEB + k)


def _tc_stage(ecat, vnp, w):
    full = lambda shape: pl.BlockSpec(shape, lambda b, j: (0,) * len(shape))
    return pl.pallas_call(
        _stage_body,
        grid=(B, NSTEP),
        in_specs=[
            pl.BlockSpec((1, TE, 8), _ecat_index),
            full((8, ED)), full((1, ED)), full((ED, ED)), full((1, ED)),
            full((ED, ED)), full((ED, 8)),
            full((3, ND)), full((ED, ND)), full((1, ND)),
            full((ND, ND)), full((1, ND)), full((ND, ND)), full((1, ND)),
            full((ND, GD)), full((ED, GD)), full((1, GD)),
            full((GD, GD)), full((1, GD)),
            full((1, GD)), full((GD, GD)), full((GD, ND)), full((GD, ED)),
            full((ND, ND)), full((ND, ND)), full((ND, ED)), full((ND, 2)),
            pl.BlockSpec((1, N, 5), lambda b, j: (b, 0, 0)),
        ],
        out_specs=[
            pl.BlockSpec((8, TE), _payT_index),
            pl.BlockSpec((1, N, 2), lambda b, j: (b, 0, 0)),
        ],
        out_shape=[
            jax.ShapeDtypeStruct((8, BE), f32),
            jax.ShapeDtypeStruct((B, N, 2), f32),
        ],
        scratch_shapes=[
            pltpu.VMEM((NEB, TE, ED), f32),
            pltpu.VMEM((NEB, TE, ED), f32),
            pltpu.VMEM((NEB, TE // EPN, ED), f32),
            pltpu.VMEM((NEB, TE // EPN, ED), f32),
            pltpu.VMEM((1, ED), f32),
        ],
    )(ecat, w["e1"], w["be1"], w["e2"], w["be2"], w["e2T"], w["e1TM"],
      w["n1np"], w["n1agg"], w["bn1"], w["n2"], w["bn2"], w["n3"], w["bn3"],
      w["g1v"], w["g1e"], w["bg1"], w["g2"], w["bg2"],
      w["outT"], w["g2T"], w["g1vT"], w["g1eT"], w["n3T"], w["n2T"],
      w["n1Tagg"], w["n1Tmom"], vnp)


def _prep_weights(params):
    w = {}
    w["e1"] = params["e1"]["W"]
    w["be1"] = params["e1"]["b"].reshape(1, ED)
    w["e2"] = params["e2"]["W"]
    w["be2"] = params["e2"]["b"].reshape(1, ED)
    w["e2T"] = params["e2"]["W"].T
    M = jnp.zeros((8, 8), f32)
    M = M.at[6, 0].set(1.0).at[7, 1].set(1.0)
    M = M.at[1, 2].set(1.0).at[2, 3].set(1.0)
    M = M.at[6, 4].set(-1.0).at[7, 5].set(-1.0)
    M = M.at[4, 6].set(1.0).at[5, 7].set(1.0)
    w["e1TM"] = params["e1"]["W"].T @ M
    wn1 = params["n1"]["W"]
    w["n1np"] = wn1[:3]
    w["n1agg"] = wn1[3:]
    w["bn1"] = params["n1"]["b"].reshape(1, ND)
    w["n2"] = params["n2"]["W"]
    w["bn2"] = params["n2"]["b"].reshape(1, ND)
    w["n3"] = params["n3"]["W"]
    w["bn3"] = params["n3"]["b"].reshape(1, ND)
    w["n2T"] = params["n2"]["W"].T
    w["n3T"] = params["n3"]["W"].T
    w["n1Tagg"] = wn1[3:].T
    w["n1Tmom"] = wn1[1:3].T
    wg1 = params["g1"]["W"]
    w["g1v"] = wg1[:ND]
    w["g1e"] = wg1[ND:]
    w["bg1"] = params["g1"]["b"].reshape(1, GD)
    w["g2"] = params["g2"]["W"]
    w["bg2"] = params["g2"]["b"].reshape(1, GD)
    w["g2T"] = params["g2"]["W"].T
    w["g1vT"] = wg1[:ND].T
    w["g1eT"] = wg1[ND:].T
    w["outT"] = params["out"]["W"][:, 0].reshape(1, GD)
    return w


def kernel(state, R_s, R_r, dt, params):
    w = _prep_weights(params)
    mc = state[:, :, 0]
    m3 = state[:, :, 0:1]
    q = state[:, :, 1:3]
    mom = state[:, :, 3:5] * m3
    v0f = jnp.concatenate([q, mom], axis=2).reshape(-1)
    table0 = jnp.concatenate([q, m3, mom], axis=2).reshape(-1)
    mcf = mc.reshape(-1)
    invmf = (1.0 / mc).reshape(-1)
    dth = (dt * 0.5).reshape(-1)
    dtf = dt.reshape(-1)
    dt6 = (dt / 6.0).reshape(-1)
    boff = (jnp.arange(B, dtype=i32) * N)[:, None]
    gs = (R_s.astype(i32) + boff).reshape(-1)
    gr = (R_r.astype(i32) + boff).reshape(-1)
    gs5 = gs * 5
    gr5 = gr * 5
    gs4 = gs * 4
    gr4 = gr * 4

    def stage(table):
        ecat = _sc_gather()(table, gs5, gr5).reshape(B, E, 8)
        payT, dnd = _tc_stage(ecat, table.reshape(B, N, 5), w)
        partials = _sc_scatter()(payT, gs4, gr4)
        return partials, dnd.reshape(-1)

    p1, dnd1 = stage(table0)
    tab2, k1 = _sc_update()(p1, dnd1, v0f, mcf, dth)
    p2, dnd2 = stage(tab2)
    tab3, k2 = _sc_update()(p2, dnd2, v0f, mcf, dth)
    p3, dnd3 = stage(tab3)
    tab4, k3 = _sc_update()(p3, dnd3, v0f, mcf, dtf)
    p4, dnd4 = stage(tab4)
    outf = _sc_final()(p4, dnd4, v0f, invmf, dt6, k1, k2, k3)
    return outf.reshape(B, N, 4)


# speedup vs baseline: 1.3059x; 1.1473x over previous
import functools

import jax
import jax.numpy as jnp
from jax import lax
from jax.experimental import pallas as pl
from jax.experimental.pallas import tpu as pltpu
from jax.experimental.pallas import tpu_sc as plsc

B, N, E = 4, 1024, 16384
ED, ND, GD = 150, 100, 100
BOX = 6.0
HALF = BOX / 2.0
EPN = E // N
BN = B * N
BE = B * E
BN4 = BN * 4

NC, NS, L = 2, 16, 16
NW = NC * NS
EPT = BE // NW
NPT = BN // NW

TE = 2048
f32 = jnp.float32
i32 = jnp.int32


@functools.cache
def _mesh():
    return plsc.VectorSubcoreMesh(core_axis_name="c", subcore_axis_name="s",
                                  num_cores=NC, num_subcores=NS)


_SC_PARAMS = pltpu.CompilerParams(needs_layout_passes=False)


def _wid():
    return lax.axis_index("s") * NC + lax.axis_index("c")


def _sp(z):
    return jnp.maximum(z, 0.0) + jnp.log(1.0 + jnp.exp(-jnp.abs(z)))


def _sig(z, t):
    r = 1.0 / (1.0 + t)
    return jnp.where(z >= 0, r, 1.0 - r)


def _wrap_pbc(q):
    q = jnp.where(q >= HALF, q - BOX, q)
    return jnp.where(q < -HALF, q + BOX, q)


def _sc_gather_body(table, idxs5, idxr5, out, tab_v, is_v, ir_v, out_v):
    wid = _wid()
    base = wid * EPT
    pltpu.sync_copy(table, tab_v)
    pltpu.sync_copy(idxs5.at[pl.ds(base, EPT)], is_v)
    pltpu.sync_copy(idxr5.at[pl.ds(base, EPT)], ir_v)
    lane = lax.iota(i32, L)

    @plsc.parallel_loop(0, EPT // L, unroll=4)
    def _(i):
        ns = is_v[pl.ds(i * L, L)]
        nr = ir_v[pl.ds(i * L, L)]
        fs = [plsc.load_gather(tab_v, [ns + c]) for c in range(5)]
        fr = [plsc.load_gather(tab_v, [nr + c]) for c in range(5)]
        d0 = fs[0] - fr[0]
        d1 = fs[1] - fr[1]
        d0 = jnp.where(d0 > HALF, d0 - BOX, d0)
        d0 = jnp.where(d0 <= -HALF, d0 + BOX, d0)
        d1 = jnp.where(d1 > HALF, d1 - BOX, d1)
        d1 = jnp.where(d1 <= -HALF, d1 + BOX, d1)
        ebase = (i * L + lane) * 8
        ch = (fs[2], fs[3], fs[4], fr[2], fr[3], fr[4], d0, d1)
        for k in range(8):
            plsc.store_scatter(out_v, [ebase + k], ch[k])

    pltpu.sync_copy(out_v, out.at[pl.ds(base * 8, EPT * 8)])


@functools.cache
def _sc_gather():
    return pl.kernel(
        _sc_gather_body,
        out_type=jax.ShapeDtypeStruct((BE * 8,), f32),
        mesh=_mesh(),
        compiler_params=_SC_PARAMS,
        scratch_types=[
            pltpu.VMEM((BN * 5,), f32),
            pltpu.VMEM((EPT,), i32),
            pltpu.VMEM((EPT,), i32),
            pltpu.VMEM((EPT * 8,), f32),
        ],
    )


def _sc_scatter_body(payT, idxs4, idxr4, partials, pay_v, is_v, ir_v, acc_v):
    wid = _wid()
    base = wid * EPT
    pltpu.sync_copy(payT.at[:, pl.ds(base, EPT)], pay_v)
    pltpu.sync_copy(idxs4.at[pl.ds(base, EPT)], is_v)
    pltpu.sync_copy(idxr4.at[pl.ds(base, EPT)], ir_v)
    lane = lax.iota(i32, L)
    zv = jnp.zeros((L,), f32)

    @plsc.parallel_loop(0, BN4 // L, unroll=8)
    def _(j):
        acc_v[pl.ds(j * L, L)] = zv

    @plsc.parallel_loop(0, EPT // L, unroll=4)
    def _(i):
        ns = is_v[pl.ds(i * L, L)]
        nr = ir_v[pl.ds(i * L, L)]
        for k in range(4):
            plsc.addupdate_scatter(acc_v, [ns + k], pay_v[k, pl.ds(i * L, L)])
        for k in range(4):
            plsc.addupdate_scatter(acc_v, [nr + k],
                                   pay_v[4 + k, pl.ds(i * L, L)])

    pltpu.sync_copy(acc_v, partials.at[pl.ds(wid * BN4, BN4)])


@functools.cache
def _sc_scatter():
    return pl.kernel(
        _sc_scatter_body,
        out_type=jax.ShapeDtypeStruct((NW * BN4,), f32),
        mesh=_mesh(),
        compiler_params=_SC_PARAMS,
        scratch_types=[
            pltpu.VMEM((8, EPT), f32),
            pltpu.VMEM((EPT,), i32),
            pltpu.VMEM((EPT,), i32),
            pltpu.VMEM((BN4,), f32),
        ],
    )


def _accum_partials(partials, acc_v, tmp_v, nbase):
    C = NPT * 4
    pltpu.sync_copy(partials.at[pl.ds(nbase * 4, C)], acc_v)

    @pl.loop(1, NW)
    def _(w):
        pltpu.sync_copy(partials.at[pl.ds(w * BN4 + nbase * 4, C)], tmp_v)

        @pl.loop(0, C // L, unroll=8)
        def _(j):
            acc_v[pl.ds(j * L, L)] = acc_v[pl.ds(j * L, L)] + tmp_v[pl.ds(j * L, L)]


def _sc_update_body(partials, dnd, v0, mc, dtc, tab_out, k_out,
                    acc_v, tmp_v, dnd_v, v0_v, mc_v, dt_v, tab_v, k_v):
    wid = _wid()
    nbase = wid * NPT
    _accum_partials(partials, acc_v, tmp_v, nbase)
    pltpu.sync_copy(dnd.at[pl.ds(nbase * 2, NPT * 2)], dnd_v)
    pltpu.sync_copy(v0.at[pl.ds(nbase * 4, NPT * 4)], v0_v)
    pltpu.sync_copy(mc.at[pl.ds(nbase, NPT)], mc_v)
    pltpu.sync_copy(dtc.at[pl.ds(nbase, NPT)], dt_v)
    lane = lax.iota(i32, L)

    @pl.loop(0, NPT // L, unroll=2)
    def _(i):
        nl = i * L + lane
        g0 = plsc.load_gather(acc_v, [nl * 4])
        g1 = plsc.load_gather(acc_v, [nl * 4 + 1])
        g2 = plsc.load_gather(acc_v, [nl * 4 + 2])
        g3 = plsc.load_gather(acc_v, [nl * 4 + 3])
        d0 = plsc.load_gather(dnd_v, [nl * 2])
        d1 = plsc.load_gather(dnd_v, [nl * 2 + 1])
        kq1 = g2 + d0
        kq2 = g3 + d1
        kp1 = -g0
        kp2 = -g1
        dtv = dt_v[pl.ds(i * L, L)]
        mcv = mc_v[pl.ds(i * L, L)]
        q1 = _wrap_pbc(plsc.load_gather(v0_v, [nl * 4]) + kq1 * dtv)
        q2 = _wrap_pbc(plsc.load_gather(v0_v, [nl * 4 + 1]) + kq2 * dtv)
        p1 = plsc.load_gather(v0_v, [nl * 4 + 2]) + kp1 * dtv
        p2 = plsc.load_gather(v0_v, [nl * 4 + 3]) + kp2 * dtv
        plsc.store_scatter(tab_v, [nl * 5], q1)
        plsc.store_scatter(tab_v, [nl * 5 + 1], q2)
        plsc.store_scatter(tab_v, [nl * 5 + 2], mcv)
        plsc.store_scatter(tab_v, [nl * 5 + 3], p1)
        plsc.store_scatter(tab_v, [nl * 5 + 4], p2)
        plsc.store_scatter(k_v, [nl * 4], kq1)
        plsc.store_scatter(k_v, [nl * 4 + 1], kq2)
        plsc.store_scatter(k_v, [nl * 4 + 2], kp1)
        plsc.store_scatter(k_v, [nl * 4 + 3], kp2)

    pltpu.sync_copy(tab_v, tab_out.at[pl.ds(nbase * 5, NPT * 5)])
    pltpu.sync_copy(k_v, k_out.at[pl.ds(nbase * 4, NPT * 4)])


@functools.cache
def _sc_update():
    return pl.kernel(
        _sc_update_body,
        out_type=(jax.ShapeDtypeStruct((BN * 5,), f32),
                  jax.ShapeDtypeStruct((BN4,), f32)),
        mesh=_mesh(),
        compiler_params=_SC_PARAMS,
        scratch_types=[
            pltpu.VMEM((NPT * 4,), f32),
            pltpu.VMEM((NPT * 4,), f32),
            pltpu.VMEM((NPT * 2,), f32),
            pltpu.VMEM((NPT * 4,), f32),
            pltpu.VMEM((NPT,), f32),
            pltpu.VMEM((NPT,), f32),
            pltpu.VMEM((NPT * 5,), f32),
            pltpu.VMEM((NPT * 4,), f32),
        ],
    )


def _sc_final_body(partials, dnd, v0, invm, dt6, k1, k2, k3, out,
                   acc_v, tmp_v, dnd_v, v0_v, im_v, dt_v, k1_v, k2_v, k3_v, o_v):
    wid = _wid()
    nbase = wid * NPT
    _accum_partials(partials, acc_v, tmp_v, nbase)
    pltpu.sync_copy(dnd.at[pl.ds(nbase * 2, NPT * 2)], dnd_v)
    pltpu.sync_copy(v0.at[pl.ds(nbase * 4, NPT * 4)], v0_v)
    pltpu.sync_copy(invm.at[pl.ds(nbase, NPT)], im_v)
    pltpu.sync_copy(dt6.at[pl.ds(nbase, NPT)], dt_v)
    pltpu.sync_copy(k1.at[pl.ds(nbase * 4, NPT * 4)], k1_v)
    pltpu.sync_copy(k2.at[pl.ds(nbase * 4, NPT * 4)], k2_v)
    pltpu.sync_copy(k3.at[pl.ds(nbase * 4, NPT * 4)], k3_v)
    lane = lax.iota(i32, L)

    @pl.loop(0, NPT // L, unroll=2)
    def _(i):
        nl = i * L + lane
        g0 = plsc.load_gather(acc_v, [nl * 4])
        g1 = plsc.load_gather(acc_v, [nl * 4 + 1])
        g2 = plsc.load_gather(acc_v, [nl * 4 + 2])
        g3 = plsc.load_gather(acc_v, [nl * 4 + 3])
        d0 = plsc.load_gather(dnd_v, [nl * 2])
        d1 = plsc.load_gather(dnd_v, [nl * 2 + 1])
        k4 = (g2 + d0, g3 + d1, -g0, -g1)
        dtv = dt_v[pl.ds(i * L, L)]
        imv = im_v[pl.ds(i * L, L)]
        for c in range(4):
            k1c = plsc.load_gather(k1_v, [nl * 4 + c])
            k2c = plsc.load_gather(k2_v, [nl * 4 + c])
            k3c = plsc.load_gather(k3_v, [nl * 4 + c])
            dy = dtv * (k1c + 2.0 * k2c + 2.0 * k3c + k4[c])
            v = plsc.load_gather(v0_v, [nl * 4 + c]) + dy
            if c < 2:
                v = _wrap_pbc(v)
            else:
                v = v * imv
            plsc.store_scatter(o_v, [nl * 4 + c], v)

    pltpu.sync_copy(o_v, out.at[pl.ds(nbase * 4, NPT * 4)])


@functools.cache
def _sc_final():
    return pl.kernel(
        _sc_final_body,
        out_type=jax.ShapeDtypeStruct((BN4,), f32),
        mesh=_mesh(),
        compiler_params=_SC_PARAMS,
        scratch_types=[
            pltpu.VMEM((NPT * 4,), f32),
            pltpu.VMEM((NPT * 4,), f32),
            pltpu.VMEM((NPT * 2,), f32),
            pltpu.VMEM((NPT * 4,), f32),
            pltpu.VMEM((NPT,), f32),
            pltpu.VMEM((NPT,), f32),
            pltpu.VMEM((NPT * 4,), f32),
            pltpu.VMEM((NPT * 4,), f32),
            pltpu.VMEM((NPT * 4,), f32),
            pltpu.VMEM((NPT * 4,), f32),
        ],
    )


NEB = E // TE
PH_NODE = NEB
NSTEP = 2 * NEB + 1


def _stage_body(ecat_ref, w1_ref, b1_ref, w2_ref, b2_ref, w2T_ref, w1TM_ref,
                wn1np_ref, wn1agg_ref, bn1_ref, wn2_ref, bn2_ref, wn3_ref,
                bn3_ref, wg1v_ref, wg1e_ref, bg1_ref, wg2_ref, bg2_ref,
                woutT_ref, wg2T_ref, wg1vT_ref, wg1eT_ref, wn3T_ref, wn2T_ref,
                wn1Tagg_ref, wn1Tmom_ref, vnp_ref,
                payT_ref, dnd_ref,
                a1_s, en_s, agg_s, dagg_s, deng_s):
    j = pl.program_id(1)

    @pl.when(j < PH_NODE)
    def _fwd():
        x = ecat_ref[0]
        z1 = jnp.dot(x, w1_ref[...], preferred_element_type=f32) + b1_ref[...]
        a1 = _sp(z1)
        z2 = jnp.dot(a1, w2_ref[...], preferred_element_type=f32) + b2_ref[...]
        en = _sp(z2)
        a1_s[j] = a1
        en_s[j] = en
        agg_s[j] = en.reshape(TE // EPN, EPN, ED).sum(axis=1)

    @pl.when(j == PH_NODE)
    def _node():
        vnp = vnp_ref[0][:, 2:5]
        agg = agg_s[...].reshape(N, ED)
        zn1 = (jnp.dot(vnp, wn1np_ref[...], preferred_element_type=f32)
               + jnp.dot(agg, wn1agg_ref[...], preferred_element_type=f32)
               + bn1_ref[...])
        t1 = jnp.exp(-jnp.abs(zn1))
        an1 = jnp.maximum(zn1, 0.0) + jnp.log(1.0 + t1)
        zn2 = (jnp.dot(an1, wn2_ref[...], preferred_element_type=f32)
               + bn2_ref[...])
        t2 = jnp.exp(-jnp.abs(zn2))
        an2 = jnp.maximum(zn2, 0.0) + jnp.log(1.0 + t2)
        zn3 = (jnp.dot(an2, wn3_ref[...], preferred_element_type=f32)
               + bn3_ref[...])
        t3 = jnp.exp(-jnp.abs(zn3))
        vn = jnp.maximum(zn3, 0.0) + jnp.log(1.0 + t3)
        vsum = vn.sum(axis=0, keepdims=True)
        esum = agg.sum(axis=0, keepdims=True)
        zg1 = (jnp.dot(vsum, wg1v_ref[...], preferred_element_type=f32)
               + jnp.dot(esum, wg1e_ref[...], preferred_element_type=f32)
               + bg1_ref[...])
        tg1 = jnp.exp(-jnp.abs(zg1))
        u1 = jnp.maximum(zg1, 0.0) + jnp.log(1.0 + tg1)
        zg2 = (jnp.dot(u1, wg2_ref[...], preferred_element_type=f32)
               + bg2_ref[...])
        tg2 = jnp.exp(-jnp.abs(zg2))
        dzg2 = woutT_ref[...] * _sig(zg2, tg2)
        du1 = jnp.dot(dzg2, wg2T_ref[...], preferred_element_type=f32)
        dzg1 = du1 * _sig(zg1, tg1)
        dgv = jnp.dot(dzg1, wg1vT_ref[...], preferred_element_type=f32)
        dge = jnp.dot(dzg1, wg1eT_ref[...], preferred_element_type=f32)
        dzn3 = dgv * _sig(zn3, t3)
        dan2 = jnp.dot(dzn3, wn3T_ref[...], preferred_element_type=f32)
        dzn2 = dan2 * _sig(zn2, t2)
        dan1 = jnp.dot(dzn2, wn2T_ref[...], preferred_element_type=f32)
        dzn1 = dan1 * _sig(zn1, t1)
        dagg_s[...] = jnp.dot(dzn1, wn1Tagg_ref[...],
                              preferred_element_type=f32).reshape(
                                  NEB, TE // EPN, ED)
        dnd_ref[0] = jnp.dot(dzn1, wn1Tmom_ref[...],
                             preferred_element_type=f32)
        deng_s[...] = dge

    @pl.when(j > PH_NODE)
    def _bwd():
        k = j - (PH_NODE + 1)
        a1 = a1_s[k]
        en = en_s[k]
        s1 = 1.0 - jnp.exp(-a1)
        s2 = 1.0 - jnp.exp(-en)
        rep = jnp.broadcast_to(dagg_s[k][:, None, :], (TE // EPN, EPN, ED))
        den = rep.reshape(TE, ED) + deng_s[...]
        dz2 = den * s2
        da1 = jnp.dot(dz2, w2T_ref[...], preferred_element_type=f32)
        dz1 = da1 * s1
        pay = jnp.dot(dz1, w1TM_ref[...], preferred_element_type=f32)
        payT_ref[...] = pay.T


def _ecat_index(b, j):
    jj = jnp.where(j < PH_NODE, j, j - (PH_NODE + 1))
    jj = jnp.maximum(jj, 0)
    return (b, jj, 0)


def _payT_index(b, j):
    k = jnp.maximum(j - (PH_NODE + 1), 0)
    return (0, b * NEB + k)


def _tc_stage(ecat, vnp, w):
    full = lambda shape: pl.BlockSpec(shape, lambda b, j: (0,) * len(shape))
    return pl.pallas_call(
        _stage_body,
        grid=(B, NSTEP),
        in_specs=[
            pl.BlockSpec((1, TE, 8), _ecat_index),
            full((8, ED)), full((1, ED)), full((ED, ED)), full((1, ED)),
            full((ED, ED)), full((ED, 8)),
            full((3, ND)), full((ED, ND)), full((1, ND)),
            full((ND, ND)), full((1, ND)), full((ND, ND)), full((1, ND)),
            full((ND, GD)), full((ED, GD)), full((1, GD)),
            full((GD, GD)), full((1, GD)),
            full((1, GD)), full((GD, GD)), full((GD, ND)), full((GD, ED)),
            full((ND, ND)), full((ND, ND)), full((ND, ED)), full((ND, 2)),
            pl.BlockSpec((1, N, 5), lambda b, j: (b, 0, 0)),
        ],
        out_specs=[
            pl.BlockSpec((8, TE), _payT_index),
            pl.BlockSpec((1, N, 2), lambda b, j: (b, 0, 0)),
        ],
        out_shape=[
            jax.ShapeDtypeStruct((8, BE), f32),
            jax.ShapeDtypeStruct((B, N, 2), f32),
        ],
        scratch_shapes=[
            pltpu.VMEM((NEB, TE, ED), f32),
            pltpu.VMEM((NEB, TE, ED), f32),
            pltpu.VMEM((NEB, TE // EPN, ED), f32),
            pltpu.VMEM((NEB, TE // EPN, ED), f32),
            pltpu.VMEM((1, ED), f32),
        ],
    )(ecat, w["e1"], w["be1"], w["e2"], w["be2"], w["e2T"], w["e1TM"],
      w["n1np"], w["n1agg"], w["bn1"], w["n2"], w["bn2"], w["n3"], w["bn3"],
      w["g1v"], w["g1e"], w["bg1"], w["g2"], w["bg2"],
      w["outT"], w["g2T"], w["g1vT"], w["g1eT"], w["n3T"], w["n2T"],
      w["n1Tagg"], w["n1Tmom"], vnp)


def _prep_weights(params):
    w = {}
    w["e1"] = params["e1"]["W"]
    w["be1"] = params["e1"]["b"].reshape(1, ED)
    w["e2"] = params["e2"]["W"]
    w["be2"] = params["e2"]["b"].reshape(1, ED)
    w["e2T"] = params["e2"]["W"].T
    M = jnp.zeros((8, 8), f32)
    M = M.at[6, 0].set(1.0).at[7, 1].set(1.0)
    M = M.at[1, 2].set(1.0).at[2, 3].set(1.0)
    M = M.at[6, 4].set(-1.0).at[7, 5].set(-1.0)
    M = M.at[4, 6].set(1.0).at[5, 7].set(1.0)
    w["e1TM"] = params["e1"]["W"].T @ M
    wn1 = params["n1"]["W"]
    w["n1np"] = wn1[:3]
    w["n1agg"] = wn1[3:]
    w["bn1"] = params["n1"]["b"].reshape(1, ND)
    w["n2"] = params["n2"]["W"]
    w["bn2"] = params["n2"]["b"].reshape(1, ND)
    w["n3"] = params["n3"]["W"]
    w["bn3"] = params["n3"]["b"].reshape(1, ND)
    w["n2T"] = params["n2"]["W"].T
    w["n3T"] = params["n3"]["W"].T
    w["n1Tagg"] = wn1[3:].T
    w["n1Tmom"] = wn1[1:3].T
    wg1 = params["g1"]["W"]
    w["g1v"] = wg1[:ND]
    w["g1e"] = wg1[ND:]
    w["bg1"] = params["g1"]["b"].reshape(1, GD)
    w["g2"] = params["g2"]["W"]
    w["bg2"] = params["g2"]["b"].reshape(1, GD)
    w["g2T"] = params["g2"]["W"].T
    w["g1vT"] = wg1[:ND].T
    w["g1eT"] = wg1[ND:].T
    w["outT"] = params["out"]["W"][:, 0].reshape(1, GD)
    return w


def kernel(state, R_s, R_r, dt, params):
    w = _prep_weights(params)
    mc = state[:, :, 0]
    m3 = state[:, :, 0:1]
    q = state[:, :, 1:3]
    mom = state[:, :, 3:5] * m3
    v0f = jnp.concatenate([q, mom], axis=2).reshape(-1)
    table0 = jnp.concatenate([q, m3, mom], axis=2).reshape(-1)
    mcf = mc.reshape(-1)
    invmf = (1.0 / mc).reshape(-1)
    dth = (dt * 0.5).reshape(-1)
    dtf = dt.reshape(-1)
    dt6 = (dt / 6.0).reshape(-1)
    boff = (jnp.arange(B, dtype=i32) * N)[:, None]
    gs = (R_s.astype(i32) + boff).reshape(-1)
    gr = (R_r.astype(i32) + boff).reshape(-1)
    gs5 = gs * 5
    gr5 = gr * 5
    gs4 = gs * 4
    gr4 = gr * 4

    def stage(table):
        ecat = _sc_gather()(table, gs5, gr5).reshape(B, E, 8)
        payT, dnd = _tc_stage(ecat, table.reshape(B, N, 5), w)
        partials = _sc_scatter()(payT, gs4, gr4)
        return partials, dnd.reshape(-1)

    p1, dnd1 = stage(table0)
    tab2, k1 = _sc_update()(p1, dnd1, v0f, mcf, dth)
    p2, dnd2 = stage(tab2)
    tab3, k2 = _sc_update()(p2, dnd2, v0f, mcf, dth)
    p3, dnd3 = stage(tab3)
    tab4, k3 = _sc_update()(p3, dnd3, v0f, mcf, dtf)
    p4, dnd4 = stage(tab4)
    outf = _sc_final()(p4, dnd4, v0f, invmf, dt6, k1, k2, k3)
    return outf.reshape(B, N, 4)


# speedup vs baseline: 1.3829x; 1.0590x over previous
import functools

import jax
import jax.numpy as jnp
from jax import lax
from jax.experimental import pallas as pl
from jax.experimental.pallas import tpu as pltpu
from jax.experimental.pallas import tpu_sc as plsc

B, N, E = 4, 1024, 16384
ED, ND, GD = 150, 100, 100
BOX = 6.0
HALF = BOX / 2.0
EPN = E // N
BN = B * N
BE = B * E
BN4 = BN * 4

NC, NS, L = 2, 16, 16
NW = NC * NS
EPT = BE // NW
NPT = BN // NW

TE = 4096
f32 = jnp.float32
i32 = jnp.int32


@functools.cache
def _mesh():
    return plsc.VectorSubcoreMesh(core_axis_name="c", subcore_axis_name="s",
                                  num_cores=NC, num_subcores=NS)


_SC_PARAMS = pltpu.CompilerParams(needs_layout_passes=False)


def _wid():
    return lax.axis_index("s") * NC + lax.axis_index("c")


def _sp(z):
    return jnp.maximum(z, 0.0) + jnp.log(1.0 + jnp.exp(-jnp.abs(z)))


def _sig(z, t):
    r = 1.0 / (1.0 + t)
    return jnp.where(z >= 0, r, 1.0 - r)


def _wrap_pbc(q):
    q = jnp.where(q >= HALF, q - BOX, q)
    return jnp.where(q < -HALF, q + BOX, q)


def _sc_gather_body(table, idxs5, idxr5, out, tab_v, is_v, ir_v, out_v):
    wid = _wid()
    base = wid * EPT
    pltpu.sync_copy(table, tab_v)
    pltpu.sync_copy(idxs5.at[pl.ds(base, EPT)], is_v)
    pltpu.sync_copy(idxr5.at[pl.ds(base, EPT)], ir_v)
    lane = lax.iota(i32, L)

    @plsc.parallel_loop(0, EPT // L, unroll=4)
    def _(i):
        ns = is_v[pl.ds(i * L, L)]
        nr = ir_v[pl.ds(i * L, L)]
        fs = [plsc.load_gather(tab_v, [ns + c]) for c in range(5)]
        fr = [plsc.load_gather(tab_v, [nr + c]) for c in range(5)]
        d0 = fs[0] - fr[0]
        d1 = fs[1] - fr[1]
        d0 = jnp.where(d0 > HALF, d0 - BOX, d0)
        d0 = jnp.where(d0 <= -HALF, d0 + BOX, d0)
        d1 = jnp.where(d1 > HALF, d1 - BOX, d1)
        d1 = jnp.where(d1 <= -HALF, d1 + BOX, d1)
        ebase = (i * L + lane) * 8
        ch = (fs[2], fs[3], fs[4], fr[2], fr[3], fr[4], d0, d1)
        for k in range(8):
            plsc.store_scatter(out_v, [ebase + k], ch[k])

    pltpu.sync_copy(out_v, out.at[pl.ds(base * 8, EPT * 8)])


@functools.cache
def _sc_gather():
    return pl.kernel(
        _sc_gather_body,
        out_type=jax.ShapeDtypeStruct((BE * 8,), f32),
        mesh=_mesh(),
        compiler_params=_SC_PARAMS,
        scratch_types=[
            pltpu.VMEM((BN * 5,), f32),
            pltpu.VMEM((EPT,), i32),
            pltpu.VMEM((EPT,), i32),
            pltpu.VMEM((EPT * 8,), f32),
        ],
    )


def _sc_scatter_body(payT, idxs4, idxr4, partials, pay_v, is_v, ir_v, acc_v):
    wid = _wid()
    base = wid * EPT
    pltpu.sync_copy(payT.at[:, pl.ds(base, EPT)], pay_v)
    pltpu.sync_copy(idxs4.at[pl.ds(base, EPT)], is_v)
    pltpu.sync_copy(idxr4.at[pl.ds(base, EPT)], ir_v)
    lane = lax.iota(i32, L)
    zv = jnp.zeros((L,), f32)

    @plsc.parallel_loop(0, BN4 // L, unroll=8)
    def _(j):
        acc_v[pl.ds(j * L, L)] = zv

    @plsc.parallel_loop(0, EPT // L, unroll=4)
    def _(i):
        ns = is_v[pl.ds(i * L, L)]
        nr = ir_v[pl.ds(i * L, L)]
        for k in range(4):
            plsc.addupdate_scatter(acc_v, [ns + k], pay_v[k, pl.ds(i * L, L)])
        for k in range(4):
            plsc.addupdate_scatter(acc_v, [nr + k],
                                   pay_v[4 + k, pl.ds(i * L, L)])

    pltpu.sync_copy(acc_v, partials.at[pl.ds(wid * BN4, BN4)])


@functools.cache
def _sc_scatter():
    return pl.kernel(
        _sc_scatter_body,
        out_type=jax.ShapeDtypeStruct((NW * BN4,), f32),
        mesh=_mesh(),
        compiler_params=_SC_PARAMS,
        scratch_types=[
            pltpu.VMEM((8, EPT), f32),
            pltpu.VMEM((EPT,), i32),
            pltpu.VMEM((EPT,), i32),
            pltpu.VMEM((BN4,), f32),
        ],
    )


def _accum_partials(partials, acc_v, tmp_v, nbase):
    C = NPT * 4
    pltpu.sync_copy(partials.at[pl.ds(nbase * 4, C)], acc_v)

    @pl.loop(1, NW)
    def _(w):
        pltpu.sync_copy(partials.at[pl.ds(w * BN4 + nbase * 4, C)], tmp_v)

        @pl.loop(0, C // L, unroll=8)
        def _(j):
            acc_v[pl.ds(j * L, L)] = acc_v[pl.ds(j * L, L)] + tmp_v[pl.ds(j * L, L)]


def _sc_update_body(partials, dnd, v0, mc, dtc, tab_out, k_out,
                    acc_v, tmp_v, dnd_v, v0_v, mc_v, dt_v, tab_v, k_v):
    wid = _wid()
    nbase = wid * NPT
    _accum_partials(partials, acc_v, tmp_v, nbase)
    pltpu.sync_copy(dnd.at[pl.ds(nbase * 2, NPT * 2)], dnd_v)
    pltpu.sync_copy(v0.at[pl.ds(nbase * 4, NPT * 4)], v0_v)
    pltpu.sync_copy(mc.at[pl.ds(nbase, NPT)], mc_v)
    pltpu.sync_copy(dtc.at[pl.ds(nbase, NPT)], dt_v)
    lane = lax.iota(i32, L)

    @pl.loop(0, NPT // L, unroll=2)
    def _(i):
        nl = i * L + lane
        g0 = plsc.load_gather(acc_v, [nl * 4])
        g1 = plsc.load_gather(acc_v, [nl * 4 + 1])
        g2 = plsc.load_gather(acc_v, [nl * 4 + 2])
        g3 = plsc.load_gather(acc_v, [nl * 4 + 3])
        d0 = plsc.load_gather(dnd_v, [nl * 2])
        d1 = plsc.load_gather(dnd_v, [nl * 2 + 1])
        kq1 = g2 + d0
        kq2 = g3 + d1
        kp1 = -g0
        kp2 = -g1
        dtv = dt_v[pl.ds(i * L, L)]
        mcv = mc_v[pl.ds(i * L, L)]
        q1 = _wrap_pbc(plsc.load_gather(v0_v, [nl * 4]) + kq1 * dtv)
        q2 = _wrap_pbc(plsc.load_gather(v0_v, [nl * 4 + 1]) + kq2 * dtv)
        p1 = plsc.load_gather(v0_v, [nl * 4 + 2]) + kp1 * dtv
        p2 = plsc.load_gather(v0_v, [nl * 4 + 3]) + kp2 * dtv
        plsc.store_scatter(tab_v, [nl * 5], q1)
        plsc.store_scatter(tab_v, [nl * 5 + 1], q2)
        plsc.store_scatter(tab_v, [nl * 5 + 2], mcv)
        plsc.store_scatter(tab_v, [nl * 5 + 3], p1)
        plsc.store_scatter(tab_v, [nl * 5 + 4], p2)
        plsc.store_scatter(k_v, [nl * 4], kq1)
        plsc.store_scatter(k_v, [nl * 4 + 1], kq2)
        plsc.store_scatter(k_v, [nl * 4 + 2], kp1)
        plsc.store_scatter(k_v, [nl * 4 + 3], kp2)

    pltpu.sync_copy(tab_v, tab_out.at[pl.ds(nbase * 5, NPT * 5)])
    pltpu.sync_copy(k_v, k_out.at[pl.ds(nbase * 4, NPT * 4)])


@functools.cache
def _sc_update():
    return pl.kernel(
        _sc_update_body,
        out_type=(jax.ShapeDtypeStruct((BN * 5,), f32),
                  jax.ShapeDtypeStruct((BN4,), f32)),
        mesh=_mesh(),
        compiler_params=_SC_PARAMS,
        scratch_types=[
            pltpu.VMEM((NPT * 4,), f32),
            pltpu.VMEM((NPT * 4,), f32),
            pltpu.VMEM((NPT * 2,), f32),
            pltpu.VMEM((NPT * 4,), f32),
            pltpu.VMEM((NPT,), f32),
            pltpu.VMEM((NPT,), f32),
            pltpu.VMEM((NPT * 5,), f32),
            pltpu.VMEM((NPT * 4,), f32),
        ],
    )


def _sc_final_body(partials, dnd, v0, invm, dt6, k1, k2, k3, out,
                   acc_v, tmp_v, dnd_v, v0_v, im_v, dt_v, k1_v, k2_v, k3_v, o_v):
    wid = _wid()
    nbase = wid * NPT
    _accum_partials(partials, acc_v, tmp_v, nbase)
    pltpu.sync_copy(dnd.at[pl.ds(nbase * 2, NPT * 2)], dnd_v)
    pltpu.sync_copy(v0.at[pl.ds(nbase * 4, NPT * 4)], v0_v)
    pltpu.sync_copy(invm.at[pl.ds(nbase, NPT)], im_v)
    pltpu.sync_copy(dt6.at[pl.ds(nbase, NPT)], dt_v)
    pltpu.sync_copy(k1.at[pl.ds(nbase * 4, NPT * 4)], k1_v)
    pltpu.sync_copy(k2.at[pl.ds(nbase * 4, NPT * 4)], k2_v)
    pltpu.sync_copy(k3.at[pl.ds(nbase * 4, NPT * 4)], k3_v)
    lane = lax.iota(i32, L)

    @pl.loop(0, NPT // L, unroll=2)
    def _(i):
        nl = i * L + lane
        g0 = plsc.load_gather(acc_v, [nl * 4])
        g1 = plsc.load_gather(acc_v, [nl * 4 + 1])
        g2 = plsc.load_gather(acc_v, [nl * 4 + 2])
        g3 = plsc.load_gather(acc_v, [nl * 4 + 3])
        d0 = plsc.load_gather(dnd_v, [nl * 2])
        d1 = plsc.load_gather(dnd_v, [nl * 2 + 1])
        k4 = (g2 + d0, g3 + d1, -g0, -g1)
        dtv = dt_v[pl.ds(i * L, L)]
        imv = im_v[pl.ds(i * L, L)]
        for c in range(4):
            k1c = plsc.load_gather(k1_v, [nl * 4 + c])
            k2c = plsc.load_gather(k2_v, [nl * 4 + c])
            k3c = plsc.load_gather(k3_v, [nl * 4 + c])
            dy = dtv * (k1c + 2.0 * k2c + 2.0 * k3c + k4[c])
            v = plsc.load_gather(v0_v, [nl * 4 + c]) + dy
            if c < 2:
                v = _wrap_pbc(v)
            else:
                v = v * imv
            plsc.store_scatter(o_v, [nl * 4 + c], v)

    pltpu.sync_copy(o_v, out.at[pl.ds(nbase * 4, NPT * 4)])


@functools.cache
def _sc_final():
    return pl.kernel(
        _sc_final_body,
        out_type=jax.ShapeDtypeStruct((BN4,), f32),
        mesh=_mesh(),
        compiler_params=_SC_PARAMS,
        scratch_types=[
            pltpu.VMEM((NPT * 4,), f32),
            pltpu.VMEM((NPT * 4,), f32),
            pltpu.VMEM((NPT * 2,), f32),
            pltpu.VMEM((NPT * 4,), f32),
            pltpu.VMEM((NPT,), f32),
            pltpu.VMEM((NPT,), f32),
            pltpu.VMEM((NPT * 4,), f32),
            pltpu.VMEM((NPT * 4,), f32),
            pltpu.VMEM((NPT * 4,), f32),
            pltpu.VMEM((NPT * 4,), f32),
        ],
    )


NEB = E // TE
PH_NODE = NEB
NSTEP = 2 * NEB + 1


def _stage_body(ecat_ref, w1_ref, b1_ref, w2_ref, b2_ref, w2T_ref, w1TM_ref,
                wn1np_ref, wn1agg_ref, bn1_ref, wn2_ref, bn2_ref, wn3_ref,
                bn3_ref, wg1v_ref, wg1e_ref, bg1_ref, wg2_ref, bg2_ref,
                woutT_ref, wg2T_ref, wg1vT_ref, wg1eT_ref, wn3T_ref, wn2T_ref,
                wn1Tagg_ref, wn1Tmom_ref, vnp_ref,
                payT_ref, dnd_ref,
                a1_s, en_s, agg_s, dagg_s, deng_s):
    j = pl.program_id(1)

    @pl.when(j < PH_NODE)
    def _fwd():
        x = ecat_ref[0]
        z1 = jnp.dot(x, w1_ref[...], preferred_element_type=f32) + b1_ref[...]
        a1 = _sp(z1)
        z2 = jnp.dot(a1, w2_ref[...], preferred_element_type=f32) + b2_ref[...]
        en = _sp(z2)
        a1_s[j] = a1
        en_s[j] = en
        agg_s[j] = en.reshape(TE // EPN, EPN, ED).sum(axis=1)

    @pl.when(j == PH_NODE)
    def _node():
        vnp = vnp_ref[0][:, 2:5]
        agg = agg_s[...].reshape(N, ED)
        zn1 = (jnp.dot(vnp, wn1np_ref[...], preferred_element_type=f32)
               + jnp.dot(agg, wn1agg_ref[...], preferred_element_type=f32)
               + bn1_ref[...])
        t1 = jnp.exp(-jnp.abs(zn1))
        an1 = jnp.maximum(zn1, 0.0) + jnp.log(1.0 + t1)
        zn2 = (jnp.dot(an1, wn2_ref[...], preferred_element_type=f32)
               + bn2_ref[...])
        t2 = jnp.exp(-jnp.abs(zn2))
        an2 = jnp.maximum(zn2, 0.0) + jnp.log(1.0 + t2)
        zn3 = (jnp.dot(an2, wn3_ref[...], preferred_element_type=f32)
               + bn3_ref[...])
        t3 = jnp.exp(-jnp.abs(zn3))
        vn = jnp.maximum(zn3, 0.0) + jnp.log(1.0 + t3)
        vsum = vn.sum(axis=0, keepdims=True)
        esum = agg.sum(axis=0, keepdims=True)
        zg1 = (jnp.dot(vsum, wg1v_ref[...], preferred_element_type=f32)
               + jnp.dot(esum, wg1e_ref[...], preferred_element_type=f32)
               + bg1_ref[...])
        tg1 = jnp.exp(-jnp.abs(zg1))
        u1 = jnp.maximum(zg1, 0.0) + jnp.log(1.0 + tg1)
        zg2 = (jnp.dot(u1, wg2_ref[...], preferred_element_type=f32)
               + bg2_ref[...])
        tg2 = jnp.exp(-jnp.abs(zg2))
        dzg2 = woutT_ref[...] * _sig(zg2, tg2)
        du1 = jnp.dot(dzg2, wg2T_ref[...], preferred_element_type=f32)
        dzg1 = du1 * _sig(zg1, tg1)
        dgv = jnp.dot(dzg1, wg1vT_ref[...], preferred_element_type=f32)
        dge = jnp.dot(dzg1, wg1eT_ref[...], preferred_element_type=f32)
        dzn3 = dgv * _sig(zn3, t3)
        dan2 = jnp.dot(dzn3, wn3T_ref[...], preferred_element_type=f32)
        dzn2 = dan2 * _sig(zn2, t2)
        dan1 = jnp.dot(dzn2, wn2T_ref[...], preferred_element_type=f32)
        dzn1 = dan1 * _sig(zn1, t1)
        dagg_s[...] = jnp.dot(dzn1, wn1Tagg_ref[...],
                              preferred_element_type=f32).reshape(
                                  NEB, TE // EPN, ED)
        dnd_ref[0] = jnp.dot(dzn1, wn1Tmom_ref[...],
                             preferred_element_type=f32)
        deng_s[...] = dge

    @pl.when(j > PH_NODE)
    def _bwd():
        k = j - (PH_NODE + 1)
        a1 = a1_s[k]
        en = en_s[k]
        s1 = 1.0 - jnp.exp(-a1)
        s2 = 1.0 - jnp.exp(-en)
        rep = jnp.broadcast_to(dagg_s[k][:, None, :], (TE // EPN, EPN, ED))
        den = rep.reshape(TE, ED) + deng_s[...]
        dz2 = den * s2
        da1 = jnp.dot(dz2, w2T_ref[...], preferred_element_type=f32)
        dz1 = da1 * s1
        pay = jnp.dot(dz1, w1TM_ref[...], preferred_element_type=f32)
        payT_ref[...] = pay.T


def _ecat_index(b, j):
    jj = jnp.where(j < PH_NODE, j, j - (PH_NODE + 1))
    jj = jnp.maximum(jj, 0)
    return (b, jj, 0)


def _payT_index(b, j):
    k = jnp.maximum(j - (PH_NODE + 1), 0)
    return (0, b * NEB + k)


def _tc_stage(ecat, vnp, w):
    full = lambda shape: pl.BlockSpec(shape, lambda b, j: (0,) * len(shape))
    return pl.pallas_call(
        _stage_body,
        grid=(B, NSTEP),
        in_specs=[
            pl.BlockSpec((1, TE, 8), _ecat_index),
            full((8, ED)), full((1, ED)), full((ED, ED)), full((1, ED)),
            full((ED, ED)), full((ED, 8)),
            full((3, ND)), full((ED, ND)), full((1, ND)),
            full((ND, ND)), full((1, ND)), full((ND, ND)), full((1, ND)),
            full((ND, GD)), full((ED, GD)), full((1, GD)),
            full((GD, GD)), full((1, GD)),
            full((1, GD)), full((GD, GD)), full((GD, ND)), full((GD, ED)),
            full((ND, ND)), full((ND, ND)), full((ND, ED)), full((ND, 2)),
            pl.BlockSpec((1, N, 5), lambda b, j: (b, 0, 0)),
        ],
        out_specs=[
            pl.BlockSpec((8, TE), _payT_index),
            pl.BlockSpec((1, N, 2), lambda b, j: (b, 0, 0)),
        ],
        out_shape=[
            jax.ShapeDtypeStruct((8, BE), f32),
            jax.ShapeDtypeStruct((B, N, 2), f32),
        ],
        scratch_shapes=[
            pltpu.VMEM((NEB, TE, ED), f32),
            pltpu.VMEM((NEB, TE, ED), f32),
            pltpu.VMEM((NEB, TE // EPN, ED), f32),
            pltpu.VMEM((NEB, TE // EPN, ED), f32),
            pltpu.VMEM((1, ED), f32),
        ],
    )(ecat, w["e1"], w["be1"], w["e2"], w["be2"], w["e2T"], w["e1TM"],
      w["n1np"], w["n1agg"], w["bn1"], w["n2"], w["bn2"], w["n3"], w["bn3"],
      w["g1v"], w["g1e"], w["bg1"], w["g2"], w["bg2"],
      w["outT"], w["g2T"], w["g1vT"], w["g1eT"], w["n3T"], w["n2T"],
      w["n1Tagg"], w["n1Tmom"], vnp)


def _prep_weights(params):
    w = {}
    w["e1"] = params["e1"]["W"]
    w["be1"] = params["e1"]["b"].reshape(1, ED)
    w["e2"] = params["e2"]["W"]
    w["be2"] = params["e2"]["b"].reshape(1, ED)
    w["e2T"] = params["e2"]["W"].T
    M = jnp.zeros((8, 8), f32)
    M = M.at[6, 0].set(1.0).at[7, 1].set(1.0)
    M = M.at[1, 2].set(1.0).at[2, 3].set(1.0)
    M = M.at[6, 4].set(-1.0).at[7, 5].set(-1.0)
    M = M.at[4, 6].set(1.0).at[5, 7].set(1.0)
    w["e1TM"] = params["e1"]["W"].T @ M
    wn1 = params["n1"]["W"]
    w["n1np"] = wn1[:3]
    w["n1agg"] = wn1[3:]
    w["bn1"] = params["n1"]["b"].reshape(1, ND)
    w["n2"] = params["n2"]["W"]
    w["bn2"] = params["n2"]["b"].reshape(1, ND)
    w["n3"] = params["n3"]["W"]
    w["bn3"] = params["n3"]["b"].reshape(1, ND)
    w["n2T"] = params["n2"]["W"].T
    w["n3T"] = params["n3"]["W"].T
    w["n1Tagg"] = wn1[3:].T
    w["n1Tmom"] = wn1[1:3].T
    wg1 = params["g1"]["W"]
    w["g1v"] = wg1[:ND]
    w["g1e"] = wg1[ND:]
    w["bg1"] = params["g1"]["b"].reshape(1, GD)
    w["g2"] = params["g2"]["W"]
    w["bg2"] = params["g2"]["b"].reshape(1, GD)
    w["g2T"] = params["g2"]["W"].T
    w["g1vT"] = wg1[:ND].T
    w["g1eT"] = wg1[ND:].T
    w["outT"] = params["out"]["W"][:, 0].reshape(1, GD)
    return w


def kernel(state, R_s, R_r, dt, params):
    w = _prep_weights(params)
    mc = state[:, :, 0]
    m3 = state[:, :, 0:1]
    q = state[:, :, 1:3]
    mom = state[:, :, 3:5] * m3
    v0f = jnp.concatenate([q, mom], axis=2).reshape(-1)
    table0 = jnp.concatenate([q, m3, mom], axis=2).reshape(-1)
    mcf = mc.reshape(-1)
    invmf = (1.0 / mc).reshape(-1)
    dth = (dt * 0.5).reshape(-1)
    dtf = dt.reshape(-1)
    dt6 = (dt / 6.0).reshape(-1)
    boff = (jnp.arange(B, dtype=i32) * N)[:, None]
    gs = (R_s.astype(i32) + boff).reshape(-1)
    gr = (R_r.astype(i32) + boff).reshape(-1)
    gs5 = gs * 5
    gr5 = gr * 5
    gs4 = gs * 4
    gr4 = gr * 4

    def stage(table):
        ecat = _sc_gather()(table, gs5, gr5).reshape(B, E, 8)
        payT, dnd = _tc_stage(ecat, table.reshape(B, N, 5), w)
        partials = _sc_scatter()(payT, gs4, gr4)
        return partials, dnd.reshape(-1)

    p1, dnd1 = stage(table0)
    tab2, k1 = _sc_update()(p1, dnd1, v0f, mcf, dth)
    p2, dnd2 = stage(tab2)
    tab3, k2 = _sc_update()(p2, dnd2, v0f, mcf, dth)
    p3, dnd3 = stage(tab3)
    tab4, k3 = _sc_update()(p3, dnd3, v0f, mcf, dtf)
    p4, dnd4 = stage(tab4)
    outf = _sc_final()(p4, dnd4, v0f, invmf, dt6, k1, k2, k3)
    return outf.reshape(B, N, 4)
